# Initial kernel scaffold; baseline (speedup 1.0000x reference)
import jax
import jax.numpy as jnp
from jax.experimental import pallas as pl


def kernel(index, data):
    raise NotImplementedError("write your pallas kernel here")


# speedup vs baseline: 1.0160x; 1.0160x over previous
import functools

import jax
import jax.numpy as jnp
from jax import lax
from jax.experimental import pallas as pl
from jax.experimental.pallas import tpu as pltpu
from jax.experimental.pallas import tpu_sc as plsc

_CYCLE_LEN = 288
_NUM_NODES = 10000
_B = 1024 * 12
_NW = 32
_BPW = _B // _NW
_CH = 8
_NCHUNK = _BPW // _CH


def _sc_gather_body(idx_hbm, table_hbm, out_hbm, idx_v, rows_v, sem):
    nc = 2
    wid = lax.axis_index("s") * nc + lax.axis_index("c")
    base = wid * _BPW
    pltpu.sync_copy(idx_hbm.at[pl.ds(base, _BPW)], idx_v)

    def chunk(g, carry):
        pltpu.async_copy(
            table_hbm.at[idx_v.at[pl.ds(g * _CH, _CH)]], rows_v, sem
        ).wait()
        pltpu.sync_copy(rows_v, out_hbm.at[pl.ds(base + g * _CH, _CH)])
        return carry

    lax.fori_loop(0, _NCHUNK, chunk, 0)


def kernel(index, data):
    idx_flat = index.reshape(-1).astype(jnp.int32)
    mesh = plsc.VectorSubcoreMesh(core_axis_name="c", subcore_axis_name="s")
    run = functools.partial(
        pl.kernel,
        mesh=mesh,
        out_type=jax.ShapeDtypeStruct((_B, _NUM_NODES), jnp.float32),
        scratch_types=[
            pltpu.VMEM((_BPW,), jnp.int32),
            pltpu.VMEM((_CH, _NUM_NODES), jnp.float32),
            pltpu.SemaphoreType.DMA,
        ],
        compiler_params=pltpu.CompilerParams(use_tc_tiling_on_sc=False),
    )(_sc_gather_body)
    out = run(idx_flat, data)
    return out.reshape(1024, 12, _NUM_NODES)


# speedup vs baseline: 1.0277x; 1.0116x over previous
import functools

import jax
import jax.numpy as jnp
from jax import lax
from jax.experimental import pallas as pl
from jax.experimental.pallas import tpu as pltpu
from jax.experimental.pallas import tpu_sc as plsc

_CYCLE_LEN = 288
_NUM_NODES = 10000
_B = 1024 * 12
_NW = 32
_BPW = _B // _NW
_CH = 6
_NCHUNK = _BPW // _CH
_T = _NCHUNK // 2


def _sc_gather_body(idx_hbm, table_hbm, out_hbm, idx_v, rows_v, gs0, gs1,
                    ws0, ws1):
    gsems = (gs0, gs1)
    wsems = (ws0, ws1)
    wid = lax.axis_index("s") * 2 + lax.axis_index("c")
    base = wid * _BPW
    pltpu.sync_copy(idx_hbm.at[wid], idx_v)

    def gather(g, b):
        return pltpu.async_copy(
            table_hbm.at[idx_v.at[g]], rows_v.at[b], gsems[b])

    def wait_gather(g, b):
        pltpu.make_async_copy(
            table_hbm.at[idx_v.at[g]], rows_v.at[b], gsems[b]).wait()

    def write(g, b):
        return pltpu.async_copy(
            rows_v.at[b], out_hbm.at[pl.ds(base + g * _CH, _CH)], wsems[b])

    def wait_write(g, b):
        pltpu.make_async_copy(
            rows_v.at[b], out_hbm.at[pl.ds(base + g * _CH, _CH)],
            wsems[b]).wait()

    def step(s, b, issue_next):
        wait_gather(s, b)
        write(s, b)
        wait_write(s - 1, 1 - b)
        if issue_next:
            gather(s + 1, 1 - b)

    gather(0, 0)
    gather(1, 1)
    wait_gather(0, 0)
    write(0, 0)
    step(1, 1, True)

    def body(t, carry):
        s = 2 * t
        step(s, 0, True)
        step(s + 1, 1, True)
        return carry

    lax.fori_loop(1, _T - 1, body, 0)

    s = _NCHUNK - 2
    step(s, 0, True)
    step(s + 1, 1, False)
    wait_write(_NCHUNK - 1, 1)


def kernel(index, data):
    idx_blocks = index.reshape(_NW, _NCHUNK, _CH).astype(jnp.int32)
    mesh = plsc.VectorSubcoreMesh(core_axis_name="c", subcore_axis_name="s")
    run = functools.partial(
        pl.kernel,
        mesh=mesh,
        out_type=jax.ShapeDtypeStruct((_B, _NUM_NODES), jnp.float32),
        scratch_types=[
            pltpu.VMEM((_NCHUNK, _CH), jnp.int32),
            pltpu.VMEM((2, _CH, _NUM_NODES), jnp.float32),
            pltpu.SemaphoreType.DMA,
            pltpu.SemaphoreType.DMA,
            pltpu.SemaphoreType.DMA,
            pltpu.SemaphoreType.DMA,
        ],
        compiler_params=pltpu.CompilerParams(use_tc_tiling_on_sc=False),
    )(_sc_gather_body)
    out = run(idx_blocks, data)
    return out.reshape(1024, 12, _NUM_NODES)


# speedup vs baseline: 1.7975x; 1.7491x over previous
import functools

import jax
import jax.numpy as jnp
from jax import lax
from jax.experimental import pallas as pl
from jax.experimental.pallas import tpu as pltpu
from jax.experimental.pallas import tpu_sc as plsc

_CYCLE_LEN = 288
_NUM_NODES = 10000
_WPAD = 10112
_WBULK = 9984
_WTAIL = _NUM_NODES - _WBULK
_NB = 1024
_NT = 12
_NW = 32
_BPW = _NB // _NW


def _sc_gather_body(idx_hbm, table_hbm, out_hbm, idx_v, g8, g4, t8, t4,
                    gs8, gs4, ws8, ws4, ts8, ts4):
    wid = lax.axis_index("s") * 2 + lax.axis_index("c")
    pltpu.sync_copy(idx_hbm.at[wid], idx_v)

    def gather8(j):
        pltpu.async_copy(table_hbm.at[idx_v.at[j, pl.ds(0, 8)]], g8, gs8)

    def gather4(j):
        pltpu.async_copy(table_hbm.at[idx_v.at[j, pl.ds(8, 4)]], g4, gs4)

    def wait_gather8(j):
        pltpu.make_async_copy(
            table_hbm.at[idx_v.at[j, pl.ds(0, 8)]], g8, gs8).wait()

    def wait_gather4(j):
        pltpu.make_async_copy(
            table_hbm.at[idx_v.at[j, pl.ds(8, 4)]], g4, gs4).wait()

    def bulk8(j, do_wait):
        bb = wid * _BPW + j
        c = pltpu.make_async_copy(
            g8.at[:, pl.ds(0, _WBULK)],
            out_hbm.at[bb, pl.ds(0, 8), pl.ds(0, _WBULK)], ws8)
        c.wait() if do_wait else c.start()

    def bulk4(j, do_wait):
        bb = wid * _BPW + j
        c = pltpu.make_async_copy(
            g4.at[:, pl.ds(0, _WBULK)],
            out_hbm.at[bb, pl.ds(8, 4), pl.ds(0, _WBULK)], ws4)
        c.wait() if do_wait else c.start()

    def tail8(j, do_wait):
        bb = wid * _BPW + j
        c = pltpu.make_async_copy(
            t8, out_hbm.at[bb, pl.ds(0, 8), pl.ds(_WBULK, _WTAIL)], ts8)
        c.wait() if do_wait else c.start()

    def tail4(j, do_wait):
        bb = wid * _BPW + j
        c = pltpu.make_async_copy(
            t4, out_hbm.at[bb, pl.ds(8, 4), pl.ds(_WBULK, _WTAIL)], ts4)
        c.wait() if do_wait else c.start()

    def step(j, first, last):
        wait_gather8(j)
        if not first:
            tail8(j - 1, True)
        for r in range(8):
            t8[r, :] = g8[r, pl.ds(_WBULK, _WTAIL)]
        bulk8(j, False)
        tail8(j, False)
        wait_gather4(j)
        if not first:
            tail4(j - 1, True)
        for r in range(4):
            t4[r, :] = g4[r, pl.ds(_WBULK, _WTAIL)]
        bulk4(j, False)
        tail4(j, False)
        bulk8(j, True)
        if not last:
            gather8(j + 1)
        bulk4(j, True)
        if not last:
            gather4(j + 1)

    gather8(0)
    gather4(0)
    step(0, True, False)
    lax.fori_loop(1, _BPW - 1, lambda j, c: (step(j, False, False), c)[1], 0)
    step(_BPW - 1, False, True)
    tail8(_BPW - 1, True)
    tail4(_BPW - 1, True)


def kernel(index, data):
    idx_blocks = index.reshape(_NW, _BPW, _NT).astype(jnp.int32)
    table = jnp.pad(data, ((0, 0), (0, _WPAD - _NUM_NODES)))
    mesh = plsc.VectorSubcoreMesh(core_axis_name="c", subcore_axis_name="s")
    run = functools.partial(
        pl.kernel,
        mesh=mesh,
        out_type=jax.ShapeDtypeStruct((_NB, _NT, _NUM_NODES), jnp.float32),
        scratch_types=[
            pltpu.VMEM((_BPW, _NT), jnp.int32),
            pltpu.VMEM((8, _WPAD), jnp.float32),
            pltpu.VMEM((4, _WPAD), jnp.float32),
            pltpu.VMEM((8, _WTAIL), jnp.float32),
            pltpu.VMEM((4, _WTAIL), jnp.float32),
            pltpu.SemaphoreType.DMA,
            pltpu.SemaphoreType.DMA,
            pltpu.SemaphoreType.DMA,
            pltpu.SemaphoreType.DMA,
            pltpu.SemaphoreType.DMA,
            pltpu.SemaphoreType.DMA,
        ],
        compiler_params=pltpu.CompilerParams(use_tc_tiling_on_sc=True),
    )(_sc_gather_body)
    return run(idx_blocks, table)
